# baseline (speedup 1.0000x reference)
import jax
import jax.numpy as jnp
from jax import lax
from jax.experimental import pallas as pl
from jax.experimental.pallas import tpu as pltpu
from jax.experimental.pallas import tpu_sc as plsc

NC = 2
NS = 16
NW = NC * NS

BATCH = 16384
N_FIELDS = 26
EMBED_DIM = 64
ROWS_W = BATCH // NW
LOOK_W = ROWS_W * N_FIELDS
BBW = ROWS_W // 128
NBUF = 4


def _body(idx_hbm, table_hbm, out_hbm, idx_v, idxT, rows_v, trans_v,
          gsem0, gsem1, gsem2, gsem3, osem0, osem1, osem2, osem3):
    wid = lax.axis_index("s") * NC + lax.axis_index("c")

    pltpu.sync_copy(idx_hbm.at[pl.ds(wid * LOOK_W, LOOK_W)], idx_v)
    iota = lax.iota(jnp.int32, 16)

    def idxt_body(b16, carry):
        bvec = b16 * 16 + iota
        for f in range(N_FIELDS):
            idxT[f, pl.ds(b16 * 16, 16)] = plsc.load_gather(
                idx_v, [bvec * N_FIELDS + f])
        return carry
    lax.fori_loop(0, ROWS_W // 16, idxt_body, 0)

    gsems = (gsem0, gsem1, gsem2, gsem3)
    osems = (osem0, osem1, osem2, osem3)

    def issue_chunk(fq, bbq, hq, b16):
        vq = idxT[fq, pl.ds(bbq * 128 + b16 * 16, 16)]
        for j in range(16):
            pltpu.make_async_copy(
                table_hbm.at[vq[j]],
                rows_v.at[hq, b16 * 16 + j],
                gsems[hq],
            ).start()

    def issue_only(fq, bbq, hq):
        def body(b16, carry):
            issue_chunk(fq, bbq, hq, b16)
            return carry
        lax.fori_loop(0, 8, body, 0)

    def fused(f, h, fq, bbq, hq, do_issue):
        def body(b16, carry):
            if do_issue:
                issue_chunk(fq, bbq, hq, b16)
            bvec = b16 * 16 + iota
            for d in range(EMBED_DIM):
                dvec = jnp.full((16,), d, jnp.int32)
                trans_v[h, d, pl.ds(b16 * 16, 16)] = plsc.load_gather(
                    rows_v.at[h], [bvec, dvec])
            return carry
        lax.fori_loop(0, 8, body, 0)

    def wait_gathers(h):
        pltpu.make_async_copy(
            table_hbm.at[pl.ds(0, 128)], rows_v.at[h], gsems[h]).wait()

    def out_descs(f, bb, h):
        BB = BBW * wid + bb
        return [
            pltpu.make_async_copy(
                trans_v.at[h, pl.ds(db * 8, 8)],
                out_hbm.at[f, db, BB],
                osems[h],
            )
            for db in range(EMBED_DIM // 8)
        ]

    issue_only(0, 0, 0)
    issue_only(0, 1, 1)
    issue_only(0, 2, 2)

    def loop_body(i, carry):
        for b in range(4):
            h = b
            hq = (b + 3) % 4
            bbq = (b + 3) % 4

            wait_gathers(h)

            @pl.when(i >= 1)
            def _():
                for d_ in out_descs(i - 1, b, h):
                    d_.wait()

            if b == 0:
                fused(i, h, i, bbq, hq, True)
            else:
                @pl.when(i < N_FIELDS - 1)
                def _():
                    fused(i, h, i + 1, bbq, hq, True)

                @pl.when(i >= N_FIELDS - 1)
                def _():
                    fused(i, h, 0, 0, hq, False)

            for d_ in out_descs(i, b, h):
                d_.start()
        return carry

    lax.fori_loop(0, N_FIELDS, loop_body, 0)

    for b in range(4):
        for d_ in out_descs(N_FIELDS - 1, b, b):
            d_.wait()


@jax.jit
def _run(indices, table):
    idx_flat = indices.reshape(BATCH * N_FIELDS)
    fn = pl.kernel(
        _body,
        out_type=jax.ShapeDtypeStruct(
            (N_FIELDS, EMBED_DIM // 8, BATCH // 128, 8, 128), jnp.float32),
        mesh=plsc.VectorSubcoreMesh(core_axis_name="c", subcore_axis_name="s"),
        compiler_params=pltpu.CompilerParams(disable_bounds_checks=True,
                                             needs_layout_passes=False),
        scratch_types=[
            pltpu.VMEM((LOOK_W,), jnp.int32),
            pltpu.VMEM((N_FIELDS, ROWS_W), jnp.int32),
            pltpu.VMEM((NBUF, 128, EMBED_DIM), jnp.float32),
            pltpu.VMEM((NBUF, EMBED_DIM, 128), jnp.float32),
        ] + [pltpu.SemaphoreType.DMA] * 8,
    )
    out5 = fn(idx_flat, table)
    return out5.transpose(2, 4, 0, 1, 3).reshape(BATCH, N_FIELDS, EMBED_DIM)


def kernel(indices, table):
    return _run(indices, table)


# speedup vs baseline: 1.6564x; 1.6564x over previous
import jax
import jax.numpy as jnp
from jax import lax
from jax.experimental import pallas as pl
from jax.experimental.pallas import tpu as pltpu
from jax.experimental.pallas import tpu_sc as plsc

NC = 2
NS = 16
NW = NC * NS

BATCH = 16384
N_FIELDS = 26
EMBED_DIM = 64
ROWS_W = BATCH // NW
LOOK_W = ROWS_W * N_FIELDS
BBW = ROWS_W // 128
NBUF = 4


def _body(idx_hbm, table_hbm, out_hbm, idx_v, idxT, rows_v, trans_v,
          gsem0, gsem1, gsem2, gsem3, osem0, osem1, osem2, osem3):
    wid = lax.axis_index("s") * NC + lax.axis_index("c")

    pltpu.sync_copy(idx_hbm.at[pl.ds(wid * LOOK_W, LOOK_W)], idx_v)
    iota = lax.iota(jnp.int32, 16)

    def idxt_body(b16, carry):
        bvec = b16 * 16 + iota
        for f in range(N_FIELDS):
            idxT[f, pl.ds(b16 * 16, 16)] = plsc.load_gather(
                idx_v, [bvec * N_FIELDS + f])
        return carry
    lax.fori_loop(0, ROWS_W // 16, idxt_body, 0)

    gsems = (gsem0, gsem1, gsem2, gsem3)
    osems = (osem0, osem1, osem2, osem3)

    def issue_chunk(fq, bbq, hq, b16):
        vq = idxT[fq, pl.ds(bbq * 128 + b16 * 16, 16)]
        for j in range(16):
            pltpu.make_async_copy(
                table_hbm.at[vq[j]],
                rows_v.at[hq, b16 * 16 + j, pl.ds(j, EMBED_DIM)],
                gsems[hq],
            ).start()

    def issue_only(fq, bbq, hq):
        def body(b16, carry):
            issue_chunk(fq, bbq, hq, b16)
            return carry
        lax.fori_loop(0, 8, body, 0)

    def fused(f, h, fq, bbq, hq, do_issue):
        def body(b16, carry):
            if do_issue:
                issue_chunk(fq, bbq, hq, b16)
            bvec = b16 * 16 + iota
            for d in range(EMBED_DIM):
                trans_v[h, d, pl.ds(b16 * 16, 16)] = plsc.load_gather(
                    rows_v.at[h], [bvec, iota + d])
            return carry
        lax.fori_loop(0, 8, body, 0)

    def wait_gathers(h):
        for _ in range(8):
            pltpu.make_async_copy(
                out_hbm.at[0, 0, 0], trans_v.at[h, pl.ds(0, 8)],
                gsems[h]).wait()

    def out_descs(f, bb, h):
        BB = BBW * wid + bb
        return [
            pltpu.make_async_copy(
                trans_v.at[h, pl.ds(db * 8, 8)],
                out_hbm.at[f, db, BB],
                osems[h],
            )
            for db in range(EMBED_DIM // 8)
        ]

    issue_only(0, 0, 0)
    issue_only(0, 1, 1)
    issue_only(0, 2, 2)

    def loop_body(i, carry):
        for b in range(4):
            h = b
            hq = (b + 3) % 4
            bbq = (b + 3) % 4

            wait_gathers(h)

            @pl.when(i >= 1)
            def _():
                for d_ in out_descs(i - 1, b, h):
                    d_.wait()

            if b == 0:
                fused(i, h, i, bbq, hq, True)
            else:
                @pl.when(i < N_FIELDS - 1)
                def _():
                    fused(i, h, i + 1, bbq, hq, True)

                @pl.when(i >= N_FIELDS - 1)
                def _():
                    fused(i, h, 0, 0, hq, False)

            for d_ in out_descs(i, b, h):
                d_.start()
        return carry

    lax.fori_loop(0, N_FIELDS, loop_body, 0)

    for b in range(4):
        for d_ in out_descs(N_FIELDS - 1, b, b):
            d_.wait()


@jax.jit
def _run(indices, table):
    idx_flat = indices.reshape(BATCH * N_FIELDS)
    fn = pl.kernel(
        _body,
        out_type=jax.ShapeDtypeStruct(
            (N_FIELDS, EMBED_DIM // 8, BATCH // 128, 8, 128), jnp.float32),
        mesh=plsc.VectorSubcoreMesh(core_axis_name="c", subcore_axis_name="s"),
        compiler_params=pltpu.CompilerParams(disable_bounds_checks=True,
                                             needs_layout_passes=False),
        scratch_types=[
            pltpu.VMEM((LOOK_W,), jnp.int32),
            pltpu.VMEM((N_FIELDS, ROWS_W), jnp.int32),
            pltpu.VMEM((NBUF, 128, 128), jnp.float32),
            pltpu.VMEM((NBUF, EMBED_DIM, 128), jnp.float32),
        ] + [pltpu.SemaphoreType.DMA] * 8,
    )
    out5 = fn(idx_flat, table)
    return out5.transpose(2, 4, 0, 1, 3).reshape(BATCH, N_FIELDS, EMBED_DIM)


def kernel(indices, table):
    return _run(indices, table)


# speedup vs baseline: 1.6565x; 1.0000x over previous
import jax
import jax.numpy as jnp
from jax import lax
from jax.experimental import pallas as pl
from jax.experimental.pallas import tpu as pltpu
from jax.experimental.pallas import tpu_sc as plsc

NC = 2
NS = 16
NW = NC * NS

BATCH = 16384
N_FIELDS = 26
EMBED_DIM = 64
ROWS_W = BATCH // NW
LOOK_W = ROWS_W * N_FIELDS
BBW = ROWS_W // 128
NBUF = 4


def _body(idx_hbm, table_hbm, out_hbm, idx_v, idxT, rows_v, trans_v,
          gsem0, gsem1, gsem2, gsem3, osem0, osem1, osem2, osem3):
    wid = lax.axis_index("s") * NC + lax.axis_index("c")

    pltpu.sync_copy(idx_hbm.at[pl.ds(wid * LOOK_W, LOOK_W)], idx_v)
    iota = lax.iota(jnp.int32, 16)

    def idxt_body(b16, carry):
        bvec = b16 * 16 + iota
        for f in range(N_FIELDS):
            idxT[f, pl.ds(b16 * 16, 16)] = plsc.load_gather(
                idx_v, [bvec * N_FIELDS + f])
        return carry
    lax.fori_loop(0, ROWS_W // 16, idxt_body, 0)

    gsems = (gsem0, gsem1, gsem2, gsem3)
    osems = (osem0, osem1, osem2, osem3)

    def issue_chunk(fq, bbq, hq, b16):
        vq = idxT[fq, pl.ds(bbq * 128 + b16 * 16, 16)]
        scalars = [vq[j] for j in range(16)]
        for j in range(16):
            pltpu.make_async_copy(
                table_hbm.at[scalars[j]],
                rows_v.at[hq, b16 * 16 + j, pl.ds(j, EMBED_DIM)],
                gsems[hq],
            ).start()

    def issue_only(fq, bbq, hq):
        def body(b16, carry):
            issue_chunk(fq, bbq, hq, b16)
            return carry
        lax.fori_loop(0, 8, body, 0)

    def fused(f, h, fq, bbq, hq, do_issue):
        def body(b16, carry):
            if do_issue:
                issue_chunk(fq, bbq, hq, b16)
            bvec = b16 * 16 + iota
            for d in range(EMBED_DIM):
                trans_v[h, d, pl.ds(b16 * 16, 16)] = plsc.load_gather(
                    rows_v.at[h], [bvec, iota + d])
            return carry
        lax.fori_loop(0, 8, body, 0)

    def wait_gathers(h):
        for _ in range(8):
            pltpu.make_async_copy(
                out_hbm.at[0, 0, 0], trans_v.at[h, pl.ds(0, 8)],
                gsems[h]).wait()

    def out_descs(f, bb, h):
        BB = BBW * wid + bb
        return [
            pltpu.make_async_copy(
                trans_v.at[h, pl.ds(db * 8, 8)],
                out_hbm.at[f, db, BB],
                osems[h],
            )
            for db in range(EMBED_DIM // 8)
        ]

    issue_only(0, 0, 0)
    issue_only(0, 1, 1)
    issue_only(0, 2, 2)

    def loop_body(i, carry):
        for b in range(4):
            h = b
            hq = (b + 3) % 4
            bbq = (b + 3) % 4

            wait_gathers(h)

            @pl.when(i >= 1)
            def _():
                for d_ in out_descs(i - 1, b, h):
                    d_.wait()

            if b == 0:
                fused(i, h, i, bbq, hq, True)
            else:
                @pl.when(i < N_FIELDS - 1)
                def _():
                    fused(i, h, i + 1, bbq, hq, True)

                @pl.when(i >= N_FIELDS - 1)
                def _():
                    fused(i, h, 0, 0, hq, False)

            for d_ in out_descs(i, b, h):
                d_.start()
        return carry

    lax.fori_loop(0, N_FIELDS, loop_body, 0)

    for b in range(4):
        for d_ in out_descs(N_FIELDS - 1, b, b):
            d_.wait()


@jax.jit
def _run(indices, table):
    idx_flat = indices.reshape(BATCH * N_FIELDS)
    fn = pl.kernel(
        _body,
        out_type=jax.ShapeDtypeStruct(
            (N_FIELDS, EMBED_DIM // 8, BATCH // 128, 8, 128), jnp.float32),
        mesh=plsc.VectorSubcoreMesh(core_axis_name="c", subcore_axis_name="s"),
        compiler_params=pltpu.CompilerParams(disable_bounds_checks=True,
                                             needs_layout_passes=False),
        scratch_types=[
            pltpu.VMEM((LOOK_W,), jnp.int32),
            pltpu.VMEM((N_FIELDS, ROWS_W), jnp.int32),
            pltpu.VMEM((NBUF, 128, 128), jnp.float32),
            pltpu.VMEM((NBUF, EMBED_DIM, 128), jnp.float32),
        ] + [pltpu.SemaphoreType.DMA] * 8,
    )
    out5 = fn(idx_flat, table)
    return out5.transpose(2, 4, 0, 1, 3).reshape(BATCH, N_FIELDS, EMBED_DIM)


def kernel(indices, table):
    return _run(indices, table)


# speedup vs baseline: 1.6567x; 1.0002x over previous
import jax
import jax.numpy as jnp
from jax import lax
from jax.experimental import pallas as pl
from jax.experimental.pallas import tpu as pltpu
from jax.experimental.pallas import tpu_sc as plsc

NC = 2
NS = 16
NW = NC * NS

BATCH = 16384
N_FIELDS = 26
EMBED_DIM = 64
ROWS_W = BATCH // NW
LOOK_W = ROWS_W * N_FIELDS
BBW = ROWS_W // 128
NBUF = 4


def _body(idx_hbm, table_hbm, out_hbm, idx_v, idxT, rows_v, trans_v,
          gsem0, gsem1, gsem2, gsem3, osem0, osem1, osem2, osem3):
    wid = lax.axis_index("s") * NC + lax.axis_index("c")

    pltpu.sync_copy(idx_hbm.at[pl.ds(wid * LOOK_W, LOOK_W)], idx_v)
    iota = lax.iota(jnp.int32, 16)

    def idxt_body(b16, carry):
        bvec = b16 * 16 + iota
        for f in range(N_FIELDS):
            idxT[f, pl.ds(b16 * 16, 16)] = plsc.load_gather(
                idx_v, [bvec * N_FIELDS + f])
        return carry
    lax.fori_loop(0, ROWS_W // 16, idxt_body, 0)

    gsems = (gsem0, gsem1, gsem2, gsem3)
    osems = (osem0, osem1, osem2, osem3)

    def issue_chunk(fq, bbq, hq, b16):
        vq = idxT[fq, pl.ds(bbq * 128 + b16 * 16, 16)]
        scalars = [vq[j] for j in range(16)]
        for j in range(16):
            pltpu.make_async_copy(
                table_hbm.at[scalars[j]],
                rows_v.at[hq, b16 * 16 + j, pl.ds(j, EMBED_DIM)],
                gsems[hq],
            ).start()

    def issue_only(fq, bbq, hq):
        def body(b16, carry):
            issue_chunk(fq, bbq, hq, b16)
            return carry
        lax.fori_loop(0, 8, body, 0)

    def fused(f, h, fq, bbq, hq, do_issue):
        def body(b16, carry):
            if do_issue:
                issue_chunk(fq, bbq, hq, b16)
            bvec = b16 * 16 + iota
            for d in range(EMBED_DIM):
                trans_v[h, d, pl.ds(b16 * 16, 16)] = plsc.load_gather(
                    rows_v.at[h], [bvec, iota + d])
            return carry
        lax.fori_loop(0, 8, body, 0)

    def wait_gathers(h):
        for _ in range(8):
            pltpu.make_async_copy(
                out_hbm.at[0, 0, 0], trans_v.at[h, pl.ds(0, 8)],
                gsems[h]).wait()

    def out_descs(f, bb, h):
        BB = BBW * wid + bb
        return [
            pltpu.make_async_copy(
                trans_v.at[h, pl.ds(db * 8, 8)],
                out_hbm.at[f, db, BB],
                osems[h],
            )
            for db in range(EMBED_DIM // 8)
        ]

    issue_only(0, 0, 0)
    issue_only(0, 1, 1)
    issue_only(0, 2, 2)

    def loop_body(i, carry):
        for b in range(4):
            h = b
            hq = (b + 3) % 4
            bbq = (b + 3) % 4

            wait_gathers(h)

            @pl.when(i >= 1)
            def _():
                for d_ in out_descs(i - 1, b, h):
                    d_.wait()

            if b == 0:
                fused(i, h, i, bbq, hq, True)
            else:
                @pl.when(i < N_FIELDS - 1)
                def _():
                    fused(i, h, i + 1, bbq, hq, True)

                @pl.when(i >= N_FIELDS - 1)
                def _():
                    fused(i, h, 0, 0, hq, False)

            for d_ in out_descs(i, b, h):
                d_.start()
        return carry

    lax.fori_loop(0, N_FIELDS, loop_body, 0)

    for b in range(4):
        for d_ in out_descs(N_FIELDS - 1, b, b):
            d_.wait()


@jax.jit
def _run(indices, table):
    idx_flat = indices.reshape(BATCH * N_FIELDS)
    table = lax.optimization_barrier(table)
    fn = pl.kernel(
        _body,
        out_type=jax.ShapeDtypeStruct(
            (N_FIELDS, EMBED_DIM // 8, BATCH // 128, 8, 128), jnp.float32),
        mesh=plsc.VectorSubcoreMesh(core_axis_name="c", subcore_axis_name="s"),
        compiler_params=pltpu.CompilerParams(disable_bounds_checks=True,
                                             needs_layout_passes=False),
        scratch_types=[
            pltpu.VMEM((LOOK_W,), jnp.int32),
            pltpu.VMEM((N_FIELDS, ROWS_W), jnp.int32),
            pltpu.VMEM((NBUF, 128, 128), jnp.float32),
            pltpu.VMEM((NBUF, EMBED_DIM, 128), jnp.float32),
        ] + [pltpu.SemaphoreType.DMA] * 8,
    )
    out5 = fn(idx_flat, table)
    return out5.transpose(2, 4, 0, 1, 3).reshape(BATCH, N_FIELDS, EMBED_DIM)


def kernel(indices, table):
    return _run(indices, table)


# speedup vs baseline: 1.6576x; 1.0005x over previous
import jax
import jax.numpy as jnp
from jax import lax
from jax.experimental import pallas as pl
from jax.experimental.pallas import tpu as pltpu
from jax.experimental.pallas import tpu_sc as plsc

NC = 2
NS = 16
NW = NC * NS

BATCH = 16384
N_FIELDS = 26
EMBED_DIM = 64
ROWS_W = BATCH // NW
LOOK_W = ROWS_W * N_FIELDS
BBW = ROWS_W // 128
NBUF = 4


def _body(idx_hbm, table_hbm, out_hbm, idx_v, idxT, rows_v, trans_v,
          gsem0, gsem1, gsem2, gsem3, osem0, osem1, osem2, osem3):
    wid = lax.axis_index("s") * NC + lax.axis_index("c")

    pltpu.sync_copy(idx_hbm.at[pl.ds(wid * LOOK_W, LOOK_W)], idx_v)
    iota = lax.iota(jnp.int32, 16)

    def idxt_body(b16, carry):
        bvec = b16 * 16 + iota
        for f in range(N_FIELDS):
            idxT[f, pl.ds(b16 * 16, 16)] = plsc.load_gather(
                idx_v, [bvec * N_FIELDS + f])
        return carry
    lax.fori_loop(0, ROWS_W // 16, idxt_body, 0)

    gsems = (gsem0, gsem1, gsem2, gsem3)
    osems = (osem0, osem1, osem2, osem3)

    def issue_chunk(fq, bbq, hq, b16):
        vq = idxT[fq, pl.ds(bbq * 128 + b16 * 16, 16)]
        scalars = [vq[j] for j in range(16)]
        for j in range(16):
            pltpu.make_async_copy(
                table_hbm.at[scalars[j]],
                rows_v.at[hq, b16 * 16 + j, pl.ds(j, EMBED_DIM)],
                gsems[hq],
            ).start()

    def issue_only(fq, bbq, hq):
        def body(b16, carry):
            issue_chunk(fq, bbq, hq, b16)
            return carry
        lax.fori_loop(0, 8, body, 0)

    def fused(f, h, fq, bbq, hq, do_issue):
        def body(b16, carry):
            if do_issue:
                issue_chunk(fq, bbq, hq, b16)
            bvec = b16 * 16 + iota
            for d in range(EMBED_DIM):
                trans_v[h, d, pl.ds(b16 * 16, 16)] = plsc.load_gather(
                    rows_v.at[h], [bvec, iota + d])
            return carry
        lax.fori_loop(0, 8, body, 0)

    def wait_gathers(h):
        for _ in range(8):
            pltpu.make_async_copy(
                out_hbm.at[0, 0, 0], trans_v.at[h, pl.ds(0, 8)],
                gsems[h]).wait()

    def out_descs(f, bb, h):
        BB = BBW * wid + bb
        return [
            pltpu.make_async_copy(
                trans_v.at[h, pl.ds(db * 8, 8)],
                out_hbm.at[f, db, BB],
                osems[h],
            )
            for db in range(EMBED_DIM // 8)
        ]

    issue_only(0, 0, 0)
    issue_only(0, 1, 1)
    issue_only(0, 2, 2)

    def loop_body(i, carry):
        for b in range(4):
            h = b
            hq = (b + 3) % 4
            bbq = (b + 3) % 4

            wait_gathers(h)

            @pl.when(i >= 1)
            def _():
                for d_ in out_descs(i - 1, b, h):
                    d_.wait()

            if b == 0:
                fused(i, h, i, bbq, hq, True)
            else:
                @pl.when(i < N_FIELDS - 1)
                def _():
                    fused(i, h, i + 1, bbq, hq, True)

                @pl.when(i >= N_FIELDS - 1)
                def _():
                    fused(i, h, 0, 0, hq, False)

            for d_ in out_descs(i, b, h):
                d_.start()
        return carry

    lax.fori_loop(0, N_FIELDS, loop_body, 0)

    for b in range(4):
        for d_ in out_descs(N_FIELDS - 1, b, b):
            d_.wait()


@jax.jit
def _run(indices, table):
    idx_flat = indices.reshape(BATCH * N_FIELDS)
    fn = pl.kernel(
        _body,
        out_type=jax.ShapeDtypeStruct(
            (N_FIELDS, EMBED_DIM // 8, BATCH // 128, 8, 128), jnp.float32),
        mesh=plsc.VectorSubcoreMesh(core_axis_name="c", subcore_axis_name="s"),
        compiler_params=pltpu.CompilerParams(disable_bounds_checks=True,
                                             needs_layout_passes=False),
        scratch_types=[
            pltpu.VMEM((LOOK_W,), jnp.int32),
            pltpu.VMEM((N_FIELDS, ROWS_W), jnp.int32),
            pltpu.VMEM((NBUF, 128, 128), jnp.float32),
            pltpu.VMEM((NBUF, EMBED_DIM, 128), jnp.float32),
        ] + [pltpu.SemaphoreType.DMA] * 8,
    )
    out5 = fn(idx_flat, table)
    return out5.transpose(2, 4, 0, 1, 3).reshape(BATCH, N_FIELDS, EMBED_DIM)


def kernel(indices, table):
    return _run(indices, table)
